# baseline (speedup 1.0000x reference)
import functools

import jax
import jax.numpy as jnp
from jax.experimental import pallas as pl
from jax.experimental.pallas import tpu as pltpu

_DILS_PARTIAL = (1, 6, 12)
_IMB = 8
_CP = 384


def _bilin_mat(in_size, out_size):
    scale = in_size / out_size
    dst = jnp.arange(out_size, dtype=jnp.float32)
    src = jnp.maximum((dst + 0.5) * scale - 0.5, 0.0)
    i0 = jnp.minimum(jnp.floor(src).astype(jnp.int32), in_size - 1)
    i1 = jnp.minimum(i0 + 1, in_size - 1)
    w1 = src - i0.astype(jnp.float32)
    w0 = 1.0 - w1
    oh0 = jax.nn.one_hot(i0, in_size, dtype=jnp.float32)
    oh1 = jax.nn.one_hot(i1, in_size, dtype=jnp.float32)
    return w0[:, None] * oh0 + w1[:, None] * oh1


def _hshift(x4, d):
    if d == 0:
        return x4
    n, h, w, c = x4.shape
    z = jnp.zeros((n, h, abs(d), c), x4.dtype)
    if d > 0:
        return jnp.concatenate([x4[:, :, d:, :], z], axis=2)
    return jnp.concatenate([z, x4[:, :, :w + d, :]], axis=2)


def _vpad(x4, p):
    n, h, w, c = x4.shape
    z = jnp.zeros((n, p, w, c), x4.dtype)
    return jnp.concatenate([z, x4, z], axis=1)


def _fused_body(xp_ref, sw_ref, ss_ref, sb_ref, wc2_ref, wb_ref, asc_ref,
                abi_ref, wout_ref, wdp_ref, ds_ref, db_ref, wh_ref, hb_ref,
                g_ref, o_ref, *, h, w):
    hw = h * w
    m = _IMB * hw
    cbr = wout_ref.shape[-1]
    cmid = wdp_ref.shape[-1]

    feat = jnp.dot(xp_ref[...].reshape(m, xp_ref.shape[-1]), sw_ref[...],
                   preferred_element_type=jnp.float32)
    feat = jnp.maximum(feat * ss_ref[...] + sb_ref[...], 0.0) \
              .astype(jnp.bfloat16)
    feat4 = feat.reshape(_IMB, h, w, _CP)

    cacc2 = jnp.dot(feat, wc2_ref[...],
                    preferred_element_type=jnp.float32)

    d_accs = []
    for bi, d in enumerate(_DILS_PARTIAL):
        shifted = {dj: _vpad(_hshift(feat4, dj), d) for dj in (-d, 0, d)}
        parts = []
        for di in (-d, 0, d):
            for dj in (-d, 0, d):
                parts.append(shifted[dj][:, d + di:d + di + h]
                             .reshape(m, _CP))
        lhs = jnp.concatenate(parts, axis=1)
        d_accs.append(jnp.dot(lhs, wb_ref[bi],
                              preferred_element_type=jnp.float32))

    accs = [cacc2[:, 0:cbr]] + d_accs + [cacc2[:, cbr:2 * cbr]]
    brs = [jnp.maximum(accs[i] * asc_ref[i:i + 1, :] + abi_ref[i:i + 1, :],
                       0.0).astype(jnp.bfloat16) for i in range(5)]
    cat = jnp.concatenate(brs, axis=1)
    y = jnp.dot(cat, wout_ref[...], preferred_element_type=jnp.float32)
    aspp = jnp.maximum(y * asc_ref[5:6, :] + abi_ref[5:6, :], 0.0) \
              .astype(jnp.bfloat16)
    aspp4 = aspp.reshape(_IMB, h, w, cmid)

    shifted = {dj: _vpad(_hshift(aspp4, dj), 1) for dj in (-1, 0, 1)}
    parts = []
    for di in (-1, 0, 1):
        for dj in (-1, 0, 1):
            parts.append(shifted[dj][:, 1 + di:1 + di + h].reshape(m, cmid))
    lhs = jnp.concatenate(parts, axis=1)
    dacc = jnp.dot(lhs, wdp_ref[...], preferred_element_type=jnp.float32)
    dec = jnp.maximum(dacc * ds_ref[...] + db_ref[...], 0.0) \
             .astype(jnp.bfloat16)

    th = jnp.dot(dec, wh_ref[...], preferred_element_type=jnp.float32)
    thT = th.reshape(_IMB, hw).T.astype(jnp.bfloat16)
    o_ref[...] = jnp.dot(g_ref[...], thT,
                         preferred_element_type=jnp.float32) + hb_ref[...]


def kernel(stem_w, stem_s, stem_b, aspp_w_taps, aspp_w_out, aspp_scale,
           aspp_bias, dec_w, dec_s, dec_b, head_w, head_b, x_nchw):
    B, C, H, W = x_nchw.shape
    P = 8
    h, w = H // P, W // P
    hw = h * w
    cin = stem_w.shape[1]
    cbr = aspp_w_out.shape[1]
    cout = aspp_w_out.shape[-1]
    cmid = dec_w.shape[-1]
    nc = head_w.shape[-1]
    OH, OW = 4 * h, 4 * w
    kp = stem_w.shape[0]

    x = x_nchw.astype(jnp.bfloat16)
    x = jnp.transpose(x, (0, 2, 3, 1)).reshape(B, h, P, w, P, C)
    x = jnp.transpose(x, (0, 1, 3, 2, 4, 5)).reshape(B, hw, P * P * C)

    cpad = _CP - cin
    sw = jnp.pad(stem_w, ((0, 0), (0, cpad)))
    ss = jnp.pad(stem_s.reshape(1, cin).astype(jnp.float32),
                 ((0, 0), (0, cpad)))
    sb = jnp.pad(stem_b.reshape(1, cin).astype(jnp.float32),
                 ((0, 0), (0, cpad)))
    wc2 = jnp.pad(
        jnp.concatenate([aspp_w_taps[0], aspp_w_taps[1 + 9 * 3 + 4]],
                        axis=-1), ((0, cpad), (0, 0)))
    wb = jnp.pad(aspp_w_taps[1:28].reshape(3, 9, cin, cbr),
                 ((0, 0), (0, 0), (0, cpad), (0, 0))) \
            .reshape(3, 9 * _CP, cbr)
    w_out_full = aspp_w_out.reshape(5 * cbr, cout)
    wdp = dec_w.reshape(9 * cmid, cmid)

    g = jnp.kron(_bilin_mat(h, OH), _bilin_mat(w, OW)) \
           .astype(jnp.bfloat16)

    body = functools.partial(_fused_body, h=h, w=w)
    out = pl.pallas_call(
        body,
        out_shape=jax.ShapeDtypeStruct((B // _IMB, OH * OW, _IMB),
                                       jnp.float32),
        grid=(B // _IMB,),
        in_specs=[
            pl.BlockSpec((_IMB, hw, P * P * C), lambda b: (b, 0, 0)),
            pl.BlockSpec((kp, _CP), lambda b: (0, 0)),
            pl.BlockSpec((1, _CP), lambda b: (0, 0)),
            pl.BlockSpec((1, _CP), lambda b: (0, 0)),
            pl.BlockSpec((_CP, 2 * cbr), lambda b: (0, 0)),
            pl.BlockSpec((3, 9 * _CP, cbr), lambda b: (0, 0, 0)),
            pl.BlockSpec((6, cout), lambda b: (0, 0)),
            pl.BlockSpec((6, cout), lambda b: (0, 0)),
            pl.BlockSpec((5 * cbr, cout), lambda b: (0, 0)),
            pl.BlockSpec((9 * cmid, cmid), lambda b: (0, 0)),
            pl.BlockSpec((1, cmid), lambda b: (0, 0)),
            pl.BlockSpec((1, cmid), lambda b: (0, 0)),
            pl.BlockSpec((cmid, nc), lambda b: (0, 0)),
            pl.BlockSpec((1, nc), lambda b: (0, 0)),
            pl.BlockSpec((OH * OW, hw), lambda b: (0, 0)),
        ],
        out_specs=pl.BlockSpec((None, OH * OW, _IMB), lambda b: (b, 0, 0)),
        compiler_params=pltpu.CompilerParams(
            dimension_semantics=("parallel",),
            vmem_limit_bytes=64 * 1024 * 1024),
    )(x, sw, ss, sb, wc2, wb, aspp_scale, aspp_bias, w_out_full, wdp,
      dec_s.reshape(1, cmid).astype(jnp.float32),
      dec_b.reshape(1, cmid).astype(jnp.float32),
      head_w, head_b, g)
    return out.transpose(0, 2, 1).reshape(B, OH, OW, nc).transpose(0, 3, 1, 2)


# speedup vs baseline: 1.0221x; 1.0221x over previous
import functools

import jax
import jax.numpy as jnp
from jax.experimental import pallas as pl
from jax.experimental.pallas import tpu as pltpu

_DILS_PARTIAL = (1, 6, 12)
_IMB = 8
_CP = 384


def _bilin_mat(in_size, out_size):
    scale = in_size / out_size
    dst = jnp.arange(out_size, dtype=jnp.float32)
    src = jnp.maximum((dst + 0.5) * scale - 0.5, 0.0)
    i0 = jnp.minimum(jnp.floor(src).astype(jnp.int32), in_size - 1)
    i1 = jnp.minimum(i0 + 1, in_size - 1)
    w1 = src - i0.astype(jnp.float32)
    w0 = 1.0 - w1
    oh0 = jax.nn.one_hot(i0, in_size, dtype=jnp.float32)
    oh1 = jax.nn.one_hot(i1, in_size, dtype=jnp.float32)
    return w0[:, None] * oh0 + w1[:, None] * oh1


def _hshift(x4, d):
    if d == 0:
        return x4
    n, h, w, c = x4.shape
    z = jnp.zeros((n, h, abs(d), c), x4.dtype)
    if d > 0:
        return jnp.concatenate([x4[:, :, d:, :], z], axis=2)
    return jnp.concatenate([z, x4[:, :, :w + d, :]], axis=2)


def _vpad(x4, p):
    n, h, w, c = x4.shape
    z = jnp.zeros((n, p, w, c), x4.dtype)
    return jnp.concatenate([z, x4, z], axis=1)


def _fused_body(xp_ref, sw_ref, ss_ref, sb_ref, wc2_ref, wb_ref, asc_ref,
                abi_ref, wout_ref, wdp_ref, ds_ref, db_ref, wh_ref, hb_ref,
                g_ref, o_ref, *, h, w):
    hw = h * w
    m = _IMB * hw
    cbr = wout_ref.shape[-1]
    cmid = wdp_ref.shape[-1]

    feat = jnp.dot(xp_ref[...].reshape(m, xp_ref.shape[-1]), sw_ref[...],
                   preferred_element_type=jnp.float32)
    feat = jnp.maximum(feat * ss_ref[...] + sb_ref[...], 0.0) \
              .astype(jnp.bfloat16)
    feat4 = feat.reshape(_IMB, h, w, _CP)

    cacc2 = jnp.dot(feat, wc2_ref[...],
                    preferred_element_type=jnp.float32)

    d_accs = []
    for bi, d in enumerate(_DILS_PARTIAL):
        shifted = {dj: _vpad(_hshift(feat4, dj), d) for dj in (-d, 0, d)}
        parts = []
        for di in (-d, 0, d):
            for dj in (-d, 0, d):
                parts.append(shifted[dj][:, d + di:d + di + h]
                             .reshape(m, _CP))
        lhs = jnp.concatenate(parts, axis=1)
        d_accs.append(jnp.dot(lhs, wb_ref[bi],
                              preferred_element_type=jnp.float32))

    accs = [cacc2[:, 0:cbr]] + d_accs + [cacc2[:, cbr:2 * cbr]]
    brs = [jnp.maximum(accs[i] * asc_ref[i:i + 1, :] + abi_ref[i:i + 1, :],
                       0.0).astype(jnp.bfloat16) for i in range(5)]
    cat = jnp.concatenate(brs, axis=1)
    y = jnp.dot(cat, wout_ref[...], preferred_element_type=jnp.float32)
    aspp = jnp.maximum(y * asc_ref[5:6, :] + abi_ref[5:6, :], 0.0) \
              .astype(jnp.bfloat16)
    aspp4 = aspp.reshape(_IMB, h, w, cmid)

    shifted = {dj: _vpad(_hshift(aspp4, dj), 1) for dj in (-1, 0, 1)}
    parts = []
    for di in (-1, 0, 1):
        for dj in (-1, 0, 1):
            parts.append(shifted[dj][:, 1 + di:1 + di + h].reshape(m, cmid))
    lhs = jnp.concatenate(parts, axis=1)
    dacc = jnp.dot(lhs, wdp_ref[...], preferred_element_type=jnp.float32)
    dec = jnp.maximum(dacc * ds_ref[...] + db_ref[...], 0.0) \
             .astype(jnp.bfloat16)

    th = jnp.dot(dec, wh_ref[...], preferred_element_type=jnp.float32)
    thT = th.reshape(_IMB, hw).T.astype(jnp.bfloat16)
    o_ref[...] = jnp.dot(g_ref[...], thT,
                         preferred_element_type=jnp.float32) + hb_ref[...]


def kernel(stem_w, stem_s, stem_b, aspp_w_taps, aspp_w_out, aspp_scale,
           aspp_bias, dec_w, dec_s, dec_b, head_w, head_b, x_nchw):
    B, C, H, W = x_nchw.shape
    P = 8
    h, w = H // P, W // P
    hw = h * w
    cin = stem_w.shape[1]
    cbr = aspp_w_out.shape[1]
    cout = aspp_w_out.shape[-1]
    cmid = dec_w.shape[-1]
    nc = head_w.shape[-1]
    OH, OW = 4 * h, 4 * w
    kp = stem_w.shape[0]

    x = x_nchw.astype(jnp.bfloat16)
    x = x.reshape(B, C, h, P, w, P).transpose(0, 1, 2, 4, 3, 5)
    x = x.reshape(B, C, hw, P * P).transpose(0, 2, 1, 3)
    x = x.reshape(B, hw, P * P * C)

    cpad = _CP - cin
    sw_r = stem_w.reshape(P, P, C, cin).transpose(2, 0, 1, 3) \
                 .reshape(kp, cin)
    sw = jnp.pad(sw_r, ((0, 0), (0, cpad)))
    ss = jnp.pad(stem_s.reshape(1, cin).astype(jnp.float32),
                 ((0, 0), (0, cpad)))
    sb = jnp.pad(stem_b.reshape(1, cin).astype(jnp.float32),
                 ((0, 0), (0, cpad)))
    wc2 = jnp.pad(
        jnp.concatenate([aspp_w_taps[0], aspp_w_taps[1 + 9 * 3 + 4]],
                        axis=-1), ((0, cpad), (0, 0)))
    wb = jnp.pad(aspp_w_taps[1:28].reshape(3, 9, cin, cbr),
                 ((0, 0), (0, 0), (0, cpad), (0, 0))) \
            .reshape(3, 9 * _CP, cbr)
    w_out_full = aspp_w_out.reshape(5 * cbr, cout)
    wdp = dec_w.reshape(9 * cmid, cmid)

    g = jnp.kron(_bilin_mat(h, OH), _bilin_mat(w, OW)) \
           .astype(jnp.bfloat16)

    body = functools.partial(_fused_body, h=h, w=w)
    out = pl.pallas_call(
        body,
        out_shape=jax.ShapeDtypeStruct((B // _IMB, OH * OW, _IMB),
                                       jnp.float32),
        grid=(B // _IMB,),
        in_specs=[
            pl.BlockSpec((_IMB, hw, P * P * C), lambda b: (b, 0, 0)),
            pl.BlockSpec((kp, _CP), lambda b: (0, 0)),
            pl.BlockSpec((1, _CP), lambda b: (0, 0)),
            pl.BlockSpec((1, _CP), lambda b: (0, 0)),
            pl.BlockSpec((_CP, 2 * cbr), lambda b: (0, 0)),
            pl.BlockSpec((3, 9 * _CP, cbr), lambda b: (0, 0, 0)),
            pl.BlockSpec((6, cout), lambda b: (0, 0)),
            pl.BlockSpec((6, cout), lambda b: (0, 0)),
            pl.BlockSpec((5 * cbr, cout), lambda b: (0, 0)),
            pl.BlockSpec((9 * cmid, cmid), lambda b: (0, 0)),
            pl.BlockSpec((1, cmid), lambda b: (0, 0)),
            pl.BlockSpec((1, cmid), lambda b: (0, 0)),
            pl.BlockSpec((cmid, nc), lambda b: (0, 0)),
            pl.BlockSpec((1, nc), lambda b: (0, 0)),
            pl.BlockSpec((OH * OW, hw), lambda b: (0, 0)),
        ],
        out_specs=pl.BlockSpec((None, OH * OW, _IMB), lambda b: (b, 0, 0)),
        compiler_params=pltpu.CompilerParams(
            dimension_semantics=("parallel",),
            vmem_limit_bytes=64 * 1024 * 1024),
    )(x, sw, ss, sb, wc2, wb, aspp_scale, aspp_bias, w_out_full, wdp,
      dec_s.reshape(1, cmid).astype(jnp.float32),
      dec_b.reshape(1, cmid).astype(jnp.float32),
      head_w, head_b, g)
    return out.transpose(0, 2, 1).reshape(B, OH, OW, nc).transpose(0, 3, 1, 2)


# speedup vs baseline: 1.0253x; 1.0031x over previous
import functools

import jax
import jax.numpy as jnp
from jax.experimental import pallas as pl
from jax.experimental.pallas import tpu as pltpu

_DILS_PARTIAL = (1, 6, 12)
_IMB = 8
_CP = 384


def _bilin_mat(in_size, out_size):
    scale = in_size / out_size
    dst = jnp.arange(out_size, dtype=jnp.float32)
    src = jnp.maximum((dst + 0.5) * scale - 0.5, 0.0)
    i0 = jnp.minimum(jnp.floor(src).astype(jnp.int32), in_size - 1)
    i1 = jnp.minimum(i0 + 1, in_size - 1)
    w1 = src - i0.astype(jnp.float32)
    w0 = 1.0 - w1
    oh0 = jax.nn.one_hot(i0, in_size, dtype=jnp.float32)
    oh1 = jax.nn.one_hot(i1, in_size, dtype=jnp.float32)
    return w0[:, None] * oh0 + w1[:, None] * oh1


def _hshift(x4, d):
    if d == 0:
        return x4
    n, h, w, c = x4.shape
    z = jnp.zeros((n, h, abs(d), c), x4.dtype)
    if d > 0:
        return jnp.concatenate([x4[:, :, d:, :], z], axis=2)
    return jnp.concatenate([z, x4[:, :, :w + d, :]], axis=2)


def _vpad(x4, p):
    n, h, w, c = x4.shape
    z = jnp.zeros((n, p, w, c), x4.dtype)
    return jnp.concatenate([z, x4, z], axis=1)


def _fused_body(xp_ref, sw_ref, ss_ref, sb_ref, wc2_ref, wb_ref, asc_ref,
                abi_ref, wout_ref, wdp_ref, ds_ref, db_ref, wh_ref, hb_ref,
                g_ref, o_ref, *, h, w):
    hw = h * w
    m = _IMB * hw
    cbr = wout_ref.shape[-1]
    cmid = wdp_ref.shape[-1]

    feat = jnp.dot(xp_ref[...].reshape(m, xp_ref.shape[-1]), sw_ref[...],
                   preferred_element_type=jnp.float32)
    feat = jnp.maximum(feat * ss_ref[...] + sb_ref[...], 0.0) \
              .astype(jnp.bfloat16)
    feat4 = feat.reshape(_IMB, h, w, _CP)

    cacc2 = jnp.dot(feat, wc2_ref[...],
                    preferred_element_type=jnp.float32)

    d_accs = []
    for bi, d in enumerate(_DILS_PARTIAL):
        shifted = {dj: _vpad(_hshift(feat4, dj), d) for dj in (-d, 0, d)}
        parts = []
        for di in (-d, 0, d):
            for dj in (-d, 0, d):
                parts.append(shifted[dj][:, d + di:d + di + h]
                             .reshape(m, _CP))
        lhs = jnp.concatenate(parts, axis=1)
        d_accs.append(jnp.dot(lhs, wb_ref[bi],
                              preferred_element_type=jnp.float32))

    accs = [cacc2[:, 0:cbr]] + d_accs + [cacc2[:, cbr:2 * cbr]]
    brs = [jnp.maximum(accs[i] * asc_ref[i:i + 1, :] + abi_ref[i:i + 1, :],
                       0.0).astype(jnp.bfloat16) for i in range(5)]
    cat = jnp.concatenate(brs, axis=1)
    y = jnp.dot(cat, wout_ref[...], preferred_element_type=jnp.float32)
    aspp = jnp.maximum(y * asc_ref[5:6, :] + abi_ref[5:6, :], 0.0) \
              .astype(jnp.bfloat16)
    aspp4 = aspp.reshape(_IMB, h, w, cmid)

    shifted = {dj: _vpad(_hshift(aspp4, dj), 1) for dj in (-1, 0, 1)}
    parts = []
    for di in (-1, 0, 1):
        for dj in (-1, 0, 1):
            parts.append(shifted[dj][:, 1 + di:1 + di + h].reshape(m, cmid))
    lhs = jnp.concatenate(parts, axis=1)
    dacc = jnp.dot(lhs, wdp_ref[...], preferred_element_type=jnp.float32)
    dec = jnp.maximum(dacc * ds_ref[...] + db_ref[...], 0.0) \
             .astype(jnp.bfloat16)

    th = jnp.dot(dec, wh_ref[...], preferred_element_type=jnp.float32)
    thT = th.reshape(_IMB, hw).T.astype(jnp.bfloat16)
    o_ref[...] = jnp.dot(g_ref[...], thT,
                         preferred_element_type=jnp.float32) + hb_ref[...]


def kernel(stem_w, stem_s, stem_b, aspp_w_taps, aspp_w_out, aspp_scale,
           aspp_bias, dec_w, dec_s, dec_b, head_w, head_b, x_nchw):
    B, C, H, W = x_nchw.shape
    P = 8
    h, w = H // P, W // P
    hw = h * w
    cin = stem_w.shape[1]
    cbr = aspp_w_out.shape[1]
    cout = aspp_w_out.shape[-1]
    cmid = dec_w.shape[-1]
    nc = head_w.shape[-1]
    OH, OW = 4 * h, 4 * w
    kp = stem_w.shape[0]

    x = x_nchw.astype(jnp.bfloat16)
    x = x.reshape(B, C, h, P, w, P).transpose(0, 1, 2, 4, 3, 5)
    x = x.reshape(B, C, hw, P * P).transpose(0, 2, 1, 3)
    x = x.reshape(B, hw, P * P * C)

    cpad = _CP - cin
    sw_r = stem_w.reshape(P, P, C, cin).transpose(2, 0, 1, 3) \
                 .reshape(kp, cin)
    sw = jnp.pad(sw_r, ((0, 0), (0, cpad)))
    ss = jnp.pad(stem_s.reshape(1, cin).astype(jnp.float32),
                 ((0, 0), (0, cpad)))
    sb = jnp.pad(stem_b.reshape(1, cin).astype(jnp.float32),
                 ((0, 0), (0, cpad)))
    wc2 = jnp.pad(
        jnp.concatenate([aspp_w_taps[0], aspp_w_taps[1 + 9 * 3 + 4]],
                        axis=-1), ((0, cpad), (0, 0)))
    wb = jnp.pad(aspp_w_taps[1:28].reshape(3, 9, cin, cbr),
                 ((0, 0), (0, 0), (0, cpad), (0, 0))) \
            .reshape(3, 9 * _CP, cbr)
    w_out_full = aspp_w_out.reshape(5 * cbr, cout)
    wdp = dec_w.reshape(9 * cmid, cmid)

    g = jnp.kron(_bilin_mat(h, OH), _bilin_mat(w, OW)) \
           .astype(jnp.bfloat16)

    body = functools.partial(_fused_body, h=h, w=w)
    out = pl.pallas_call(
        body,
        out_shape=jax.ShapeDtypeStruct((B // _IMB, OH * OW, _IMB),
                                       jnp.float32),
        grid=(B // _IMB,),
        in_specs=[
            pl.BlockSpec((_IMB, hw, P * P * C), lambda b: (b, 0, 0)),
            pl.BlockSpec((kp, _CP), lambda b: (0, 0)),
            pl.BlockSpec((1, _CP), lambda b: (0, 0)),
            pl.BlockSpec((1, _CP), lambda b: (0, 0)),
            pl.BlockSpec((_CP, 2 * cbr), lambda b: (0, 0)),
            pl.BlockSpec((3, 9 * _CP, cbr), lambda b: (0, 0, 0)),
            pl.BlockSpec((6, cout), lambda b: (0, 0)),
            pl.BlockSpec((6, cout), lambda b: (0, 0)),
            pl.BlockSpec((5 * cbr, cout), lambda b: (0, 0)),
            pl.BlockSpec((9 * cmid, cmid), lambda b: (0, 0)),
            pl.BlockSpec((1, cmid), lambda b: (0, 0)),
            pl.BlockSpec((1, cmid), lambda b: (0, 0)),
            pl.BlockSpec((cmid, nc), lambda b: (0, 0)),
            pl.BlockSpec((1, nc), lambda b: (0, 0)),
            pl.BlockSpec((OH * OW, hw), lambda b: (0, 0)),
        ],
        out_specs=pl.BlockSpec((None, OH * OW, _IMB), lambda b: (b, 0, 0)),
        compiler_params=pltpu.CompilerParams(
            dimension_semantics=("parallel",),
            vmem_limit_bytes=56 * 1024 * 1024),
    )(x, sw, ss, sb, wc2, wb, aspp_scale, aspp_bias, w_out_full, wdp,
      dec_s.reshape(1, cmid).astype(jnp.float32),
      dec_b.reshape(1, cmid).astype(jnp.float32),
      head_w, head_b, g)
    return out.transpose(0, 2, 1).reshape(B, OH, OW, nc).transpose(0, 3, 1, 2)


# speedup vs baseline: 1.0982x; 1.0711x over previous
import functools

import jax
import jax.numpy as jnp
from jax.experimental import pallas as pl
from jax.experimental.pallas import tpu as pltpu

_DILS_PARTIAL = (1, 6, 12)
_IMB = 8
_CP = 320


def _bilin_mat(in_size, out_size):
    scale = in_size / out_size
    dst = jnp.arange(out_size, dtype=jnp.float32)
    src = jnp.maximum((dst + 0.5) * scale - 0.5, 0.0)
    i0 = jnp.minimum(jnp.floor(src).astype(jnp.int32), in_size - 1)
    i1 = jnp.minimum(i0 + 1, in_size - 1)
    w1 = src - i0.astype(jnp.float32)
    w0 = 1.0 - w1
    oh0 = jax.nn.one_hot(i0, in_size, dtype=jnp.float32)
    oh1 = jax.nn.one_hot(i1, in_size, dtype=jnp.float32)
    return w0[:, None] * oh0 + w1[:, None] * oh1


def _hshift(x4, d):
    if d == 0:
        return x4
    n, h, w, c = x4.shape
    z = jnp.zeros((n, h, abs(d), c), x4.dtype)
    if d > 0:
        return jnp.concatenate([x4[:, :, d:, :], z], axis=2)
    return jnp.concatenate([z, x4[:, :, :w + d, :]], axis=2)


def _vpad(x4, p):
    n, h, w, c = x4.shape
    z = jnp.zeros((n, p, w, c), x4.dtype)
    return jnp.concatenate([z, x4, z], axis=1)


def _fused_body(xp_ref, sw_ref, ss_ref, sb_ref, wc2_ref, wb_ref, asc_ref,
                abi_ref, wout_ref, wdp_ref, ds_ref, db_ref, wh_ref, hb_ref,
                g_ref, o_ref, *, h, w):
    hw = h * w
    m = _IMB * hw
    cbr = wout_ref.shape[-1]
    cmid = wdp_ref.shape[-1]

    feat = jnp.dot(xp_ref[...].reshape(m, xp_ref.shape[-1]), sw_ref[...],
                   preferred_element_type=jnp.float32)
    feat = jnp.maximum(feat * ss_ref[...] + sb_ref[...], 0.0) \
              .astype(jnp.bfloat16)
    feat4 = feat.reshape(_IMB, h, w, _CP)

    cacc2 = jnp.dot(feat, wc2_ref[...],
                    preferred_element_type=jnp.float32)

    d_accs = []
    for bi, d in enumerate(_DILS_PARTIAL):
        shifted = {dj: _vpad(_hshift(feat4, dj), d) for dj in (-d, 0, d)}
        parts = []
        for di in (-d, 0, d):
            for dj in (-d, 0, d):
                parts.append(shifted[dj][:, d + di:d + di + h]
                             .reshape(m, _CP))
        lhs = jnp.concatenate(parts, axis=1)
        d_accs.append(jnp.dot(lhs, wb_ref[bi],
                              preferred_element_type=jnp.float32))

    accs = [cacc2[:, 0:cbr]] + d_accs + [cacc2[:, cbr:2 * cbr]]
    brs = [jnp.maximum(accs[i] * asc_ref[i:i + 1, :] + abi_ref[i:i + 1, :],
                       0.0).astype(jnp.bfloat16) for i in range(5)]
    cat = jnp.concatenate(brs, axis=1)
    y = jnp.dot(cat, wout_ref[...], preferred_element_type=jnp.float32)
    aspp = jnp.maximum(y * asc_ref[5:6, :] + abi_ref[5:6, :], 0.0) \
              .astype(jnp.bfloat16)
    aspp4 = aspp.reshape(_IMB, h, w, cmid)

    shifted = {dj: _vpad(_hshift(aspp4, dj), 1) for dj in (-1, 0, 1)}
    parts = []
    for di in (-1, 0, 1):
        for dj in (-1, 0, 1):
            parts.append(shifted[dj][:, 1 + di:1 + di + h].reshape(m, cmid))
    lhs = jnp.concatenate(parts, axis=1)
    dacc = jnp.dot(lhs, wdp_ref[...], preferred_element_type=jnp.float32)
    dec = jnp.maximum(dacc * ds_ref[...] + db_ref[...], 0.0) \
             .astype(jnp.bfloat16)

    th = jnp.dot(dec, wh_ref[...], preferred_element_type=jnp.float32)
    thT = th.reshape(_IMB, hw).T.astype(jnp.bfloat16)
    o_ref[...] = jnp.dot(g_ref[...], thT,
                         preferred_element_type=jnp.float32) + hb_ref[...]


def kernel(stem_w, stem_s, stem_b, aspp_w_taps, aspp_w_out, aspp_scale,
           aspp_bias, dec_w, dec_s, dec_b, head_w, head_b, x_nchw):
    B, C, H, W = x_nchw.shape
    P = 8
    h, w = H // P, W // P
    hw = h * w
    cin = stem_w.shape[1]
    cbr = aspp_w_out.shape[1]
    cout = aspp_w_out.shape[-1]
    cmid = dec_w.shape[-1]
    nc = head_w.shape[-1]
    OH, OW = 4 * h, 4 * w
    kp = stem_w.shape[0]

    x = x_nchw.astype(jnp.bfloat16)
    x = x.reshape(B, C, h, P, w, P).transpose(0, 1, 2, 4, 3, 5)
    x = x.reshape(B, C, hw, P * P).transpose(0, 2, 1, 3)
    x = x.reshape(B, hw, P * P * C)

    cpad = _CP - cin
    sw_r = stem_w.reshape(P, P, C, cin).transpose(2, 0, 1, 3) \
                 .reshape(kp, cin)
    sw = jnp.pad(sw_r, ((0, 0), (0, cpad)))
    ss = jnp.pad(stem_s.reshape(1, cin).astype(jnp.float32),
                 ((0, 0), (0, cpad)))
    sb = jnp.pad(stem_b.reshape(1, cin).astype(jnp.float32),
                 ((0, 0), (0, cpad)))
    wc2 = jnp.pad(
        jnp.concatenate([aspp_w_taps[0], aspp_w_taps[1 + 9 * 3 + 4]],
                        axis=-1), ((0, cpad), (0, 0)))
    wb = jnp.pad(aspp_w_taps[1:28].reshape(3, 9, cin, cbr),
                 ((0, 0), (0, 0), (0, cpad), (0, 0))) \
            .reshape(3, 9 * _CP, cbr)
    w_out_full = aspp_w_out.reshape(5 * cbr, cout)
    wdp = dec_w.reshape(9 * cmid, cmid)

    g = jnp.kron(_bilin_mat(h, OH), _bilin_mat(w, OW)) \
           .astype(jnp.bfloat16)

    body = functools.partial(_fused_body, h=h, w=w)
    out = pl.pallas_call(
        body,
        out_shape=jax.ShapeDtypeStruct((B // _IMB, OH * OW, _IMB),
                                       jnp.float32),
        grid=(B // _IMB,),
        in_specs=[
            pl.BlockSpec((_IMB, hw, P * P * C), lambda b: (b, 0, 0)),
            pl.BlockSpec((kp, _CP), lambda b: (0, 0)),
            pl.BlockSpec((1, _CP), lambda b: (0, 0)),
            pl.BlockSpec((1, _CP), lambda b: (0, 0)),
            pl.BlockSpec((_CP, 2 * cbr), lambda b: (0, 0)),
            pl.BlockSpec((3, 9 * _CP, cbr), lambda b: (0, 0, 0)),
            pl.BlockSpec((6, cout), lambda b: (0, 0)),
            pl.BlockSpec((6, cout), lambda b: (0, 0)),
            pl.BlockSpec((5 * cbr, cout), lambda b: (0, 0)),
            pl.BlockSpec((9 * cmid, cmid), lambda b: (0, 0)),
            pl.BlockSpec((1, cmid), lambda b: (0, 0)),
            pl.BlockSpec((1, cmid), lambda b: (0, 0)),
            pl.BlockSpec((cmid, nc), lambda b: (0, 0)),
            pl.BlockSpec((1, nc), lambda b: (0, 0)),
            pl.BlockSpec((OH * OW, hw), lambda b: (0, 0)),
        ],
        out_specs=pl.BlockSpec((None, OH * OW, _IMB), lambda b: (b, 0, 0)),
        compiler_params=pltpu.CompilerParams(
            dimension_semantics=("parallel",),
            vmem_limit_bytes=56 * 1024 * 1024),
    )(x, sw, ss, sb, wc2, wb, aspp_scale, aspp_bias, w_out_full, wdp,
      dec_s.reshape(1, cmid).astype(jnp.float32),
      dec_b.reshape(1, cmid).astype(jnp.float32),
      head_w, head_b, g)
    return out.transpose(0, 2, 1).reshape(B, OH, OW, nc).transpose(0, 3, 1, 2)
